# baseline (speedup 1.0000x reference)
import functools

import jax
import jax.numpy as jnp
from jax import lax
from jax.experimental import pallas as pl
from jax.experimental.pallas import tpu as pltpu
from jax.experimental.pallas import tpu_sc as plsc

A = 10001
E = 320000
MAXB = 32
H = 128
NM = 100
L = 100
A_PAD = 10240
NMP = 128

_NW = 32
_APW = A_PAD // _NW
_GA = 4
_NGA = _APW // _GA
_EPW = E // _NW
_GB = 128
_NGB = _EPW // _GB
_GBR = _EPW - _NGB * _GB
_NBUF = 4


def _mm(x, wt, block_rows, relu):
    n, k = x.shape
    h = wt.shape[1]

    def body(x_ref, w_ref, o_ref):
        y = jnp.dot(x_ref[...], w_ref[...], preferred_element_type=jnp.float32)
        if relu:
            y = jnp.maximum(y, 0.0)
        o_ref[...] = y

    return pl.pallas_call(
        body,
        grid=(n // block_rows,),
        in_specs=[pl.BlockSpec((block_rows, k), lambda i: (i, 0)),
                  pl.BlockSpec((k, h), lambda i: (0, 0))],
        out_specs=pl.BlockSpec((block_rows, h), lambda i: (i, 0)),
        out_shape=jax.ShapeDtypeStruct((n, h), jnp.float32),
    )(x, wt)


def _add(msg, agg):
    blk = 1024

    def body(m_ref, a_ref, o_ref):
        o_ref[...] = m_ref[...] + a_ref[...]

    return pl.pallas_call(
        body,
        grid=(A_PAD // blk,),
        in_specs=[pl.BlockSpec((blk, H), lambda i: (i, 0)),
                  pl.BlockSpec((blk, H), lambda i: (i, 0))],
        out_specs=pl.BlockSpec((blk, H), lambda i: (i, 0)),
        out_shape=jax.ShapeDtypeStruct((A_PAD, H), jnp.float32),
    )(msg, agg)


def _bond_mm(pre, ib, wt):
    blk = 1280

    def body(p_ref, i_ref, w_ref, o_ref):
        o_ref[...] = jnp.maximum(
            i_ref[...] + jnp.dot(p_ref[...], w_ref[...],
                                 preferred_element_type=jnp.float32), 0.0)

    return pl.pallas_call(
        body,
        grid=(E // blk,),
        in_specs=[pl.BlockSpec((blk, H), lambda i: (i, 0)),
                  pl.BlockSpec((blk, H), lambda i: (i, 0)),
                  pl.BlockSpec((H, H), lambda i: (0, 0))],
        out_specs=pl.BlockSpec((blk, H), lambda i: (i, 0)),
        out_shape=jax.ShapeDtypeStruct((E, H), jnp.float32),
    )(pre, ib, wt)


def _mm3(agg3, msg2, ia, w0, w1, w2, bias):
    blk = 1024

    def body(a_ref, m_ref, i_ref, w0_ref, w1_ref, w2_ref, b_ref, hid_ref, msg_ref):
        h = (jnp.dot(a_ref[...], w0_ref[...], preferred_element_type=jnp.float32)
             + jnp.dot(m_ref[...], w1_ref[...], preferred_element_type=jnp.float32)
             + jnp.dot(i_ref[...], w2_ref[...], preferred_element_type=jnp.float32))
        hid_ref[...] = h
        msg_ref[...] = jnp.maximum(h + b_ref[...], 0.0)

    return pl.pallas_call(
        body,
        grid=(A_PAD // blk,),
        in_specs=[pl.BlockSpec((blk, H), lambda i: (i, 0)),
                  pl.BlockSpec((blk, H), lambda i: (i, 0)),
                  pl.BlockSpec((blk, H), lambda i: (i, 0)),
                  pl.BlockSpec((H, H), lambda i: (0, 0)),
                  pl.BlockSpec((H, H), lambda i: (0, 0)),
                  pl.BlockSpec((H, H), lambda i: (0, 0)),
                  pl.BlockSpec((1, H), lambda i: (0, 0))],
        out_specs=[pl.BlockSpec((blk, H), lambda i: (i, 0)),
                   pl.BlockSpec((blk, H), lambda i: (i, 0))],
        out_shape=[jax.ShapeDtypeStruct((A_PAD, H), jnp.float32),
                   jax.ShapeDtypeStruct((A_PAD, H), jnp.float32)],
    )(agg3, msg2, ia, w0, w1, w2, bias)


def _h0_kernel(hid3):
    def body(h_ref, o_ref):
        o_ref[...] = jnp.max(h_ref[...], axis=1)

    return pl.pallas_call(
        body,
        out_shape=jax.ShapeDtypeStruct((NM, H), jnp.float32),
    )(hid3)


def _gru_kernel(seq, h0p, m0, wif, whf, bif, bhf, wib, whb, bib, bhb, wo1, wo2, bo):
    def body(seq_ref, h0_ref, m0_ref, wif_ref, whf_ref, bif_ref, bhf_ref,
             wib_ref, whb_ref, bib_ref, bhb_ref, wo1_ref, wo2_ref, bo_ref,
             out_ref, r0_ref):
        out_ref[...] = jnp.zeros_like(out_ref)
        wifv = wif_ref[...]
        whfv = whf_ref[...]
        wibv = wib_ref[...]
        whbv = whb_ref[...]
        bifv = bif_ref[...]
        bhfv = bhf_ref[...]
        bibv = bib_ref[...]
        bhbv = bhb_ref[...]
        wo1v = wo1_ref[...]
        wo2v = wo2_ref[...]

        def step(h, x, wih, whh, bi, bh):
            gi = jnp.dot(x, wih, preferred_element_type=jnp.float32) + bi
            gh = jnp.dot(h, whh, preferred_element_type=jnp.float32) + bh
            r = jax.nn.sigmoid(gi[:, :H] + gh[:, :H])
            z = jax.nn.sigmoid(gi[:, H:2 * H] + gh[:, H:2 * H])
            n = jnp.tanh(gi[:, 2 * H:] + r * gh[:, 2 * H:])
            return (1.0 - z) * n + z * h

        def loop(t, carry):
            hf, hb = carry
            xf = seq_ref[pl.ds(t * NMP, NMP), :]
            xb = seq_ref[pl.ds((L - 1 - t) * NMP, NMP), :]
            hf = step(hf, xf, wifv, whfv, bifv, bhfv)
            hb = step(hb, xb, wibv, whbv, bibv, bhbv)
            out_ref[pl.ds(t * NMP, NMP), :] += jnp.dot(
                hf, wo1v, preferred_element_type=jnp.float32)
            out_ref[pl.ds((L - 1 - t) * NMP, NMP), :] += jnp.dot(
                hb, wo2v, preferred_element_type=jnp.float32)
            return (hf, hb)

        h0 = h0_ref[...]
        lax.fori_loop(0, L, loop, (h0, h0))
        out_ref[...] = jnp.maximum(out_ref[...] + bo_ref[...], 0.0)
        r0_ref[...] = jnp.maximum(
            jnp.dot(m0_ref[...], wo1v + wo2v, preferred_element_type=jnp.float32)
            + bo_ref[...], 0.0)

    return pl.pallas_call(
        body,
        out_shape=[jax.ShapeDtypeStruct((L * NMP, H), jnp.float32),
                   jax.ShapeDtypeStruct((8, H), jnp.float32)],
    )(seq, h0p, m0, wif, whf, bif, bhf, wib, whb, bib, bhb, wo1, wo2, bo)


def _sc_mesh():
    return plsc.VectorSubcoreMesh(core_axis_name="c", subcore_axis_name="s")


def _sc_agg(mb, a2b_flat):
    @functools.partial(
        pl.kernel,
        mesh=_sc_mesh(),
        out_type=jax.ShapeDtypeStruct((A_PAD, H), jnp.float32),
        scratch_types=[
            pltpu.VMEM((_APW * MAXB,), jnp.int32),
            pltpu.VMEM((_NBUF, _GA * MAXB, H), jnp.float32),
            pltpu.VMEM((_APW, H), jnp.float32),
            pltpu.SemaphoreType.DMA,
        ],
    )
    def k(mb_ref, idx_ref, out_ref, idx_v, rows_v, out_v, sem):
        wid = lax.axis_index("s") * 2 + lax.axis_index("c")
        pltpu.sync_copy(idx_ref.at[pl.ds(wid * (_APW * MAXB), _APW * MAXB)], idx_v)

        def batch(k_, carry):
            g0 = k_ * _NBUF
            handles = [
                pltpu.async_copy(
                    mb_ref.at[idx_v.at[pl.ds((g0 + b) * (_GA * MAXB), _GA * MAXB)]],
                    rows_v.at[b], sem)
                for b in range(_NBUF)
            ]
            for h in handles:
                h.wait()
            for b in range(_NBUF):
                def atom(i, c2, b=b):
                    base = i * MAXB
                    for c in range(H // 16):
                        sl = pl.ds(c * 16, 16)
                        v = rows_v[b, base, sl]
                        s = v
                        m = v
                        for j in range(1, MAXB):
                            v = rows_v[b, base + j, sl]
                            s = s + v
                            m = jnp.maximum(m, v)
                        out_v[(g0 + b) * _GA + i, sl] = s * m
                    return c2

                lax.fori_loop(0, _GA, atom, 0)
            return carry

        lax.fori_loop(0, _NGA // _NBUF, batch, 0)
        pltpu.sync_copy(out_v, out_ref.at[pl.ds(wid * _APW, _APW)])

    return k(mb, a2b_flat)


def _sc_pre(t_tab, u_tab, b2a, b2revb):
    @functools.partial(
        pl.kernel,
        mesh=_sc_mesh(),
        out_type=jax.ShapeDtypeStruct((E, H), jnp.float32),
        scratch_types=[
            pltpu.VMEM((_EPW,), jnp.int32),
            pltpu.VMEM((_EPW,), jnp.int32),
            pltpu.VMEM((2, _GB, H), jnp.float32),
            pltpu.VMEM((2, _GB, H), jnp.float32),
            pltpu.VMEM((2, _GB, H), jnp.float32),
            pltpu.SemaphoreType.DMA,
        ],
    )
    def k(t_ref, u_ref, ba_ref, br_ref, out_ref,
          ia_v, ir_v, t_v, u_v, o_v, sem):
        wid = lax.axis_index("s") * 2 + lax.axis_index("c")
        base = wid * _EPW
        pltpu.sync_copy(ba_ref.at[pl.ds(base, _EPW)], ia_v)
        pltpu.sync_copy(br_ref.at[pl.ds(base, _EPW)], ir_v)

        def compute_rows(s_, n_rows):
            def row(r, c2):
                for cc in range(H // 16):
                    sl = pl.ds(cc * 16, 16)
                    o_v[s_, r, sl] = t_v[s_, r, sl] - u_v[s_, r, sl]
                return c2

            lax.fori_loop(0, n_rows, row, 0)

        def batch(k_, carry):
            g0 = k_ * 2
            handles = []
            for s_ in range(2):
                g = g0 + s_
                handles.append(pltpu.async_copy(
                    t_ref.at[ia_v.at[pl.ds(g * _GB, _GB)]], t_v.at[s_], sem))
                handles.append(pltpu.async_copy(
                    u_ref.at[ir_v.at[pl.ds(g * _GB, _GB)]], u_v.at[s_], sem))
            for h in handles:
                h.wait()
            for s_ in range(2):
                compute_rows(s_, _GB)
                pltpu.sync_copy(o_v.at[s_],
                                out_ref.at[pl.ds(base + (g0 + s_) * _GB, _GB)])
            return carry

        lax.fori_loop(0, _NGB // 2, batch, 0)

        roff = _NGB * _GB
        ct = pltpu.async_copy(
            t_ref.at[ia_v.at[pl.ds(roff, _GBR)]], t_v.at[0, pl.ds(0, _GBR)], sem)
        cu = pltpu.async_copy(
            u_ref.at[ir_v.at[pl.ds(roff, _GBR)]], u_v.at[0, pl.ds(0, _GBR)], sem)
        ct.wait()
        cu.wait()
        compute_rows(0, _GBR)
        pltpu.sync_copy(o_v.at[0, pl.ds(0, _GBR)],
                        out_ref.at[pl.ds(base + roff, _GBR)])

    return k(t_tab, u_tab, b2a, b2revb)


def kernel(f_atoms, f_bonds, a2b, b2a, b2revb, a_scope, W_i_atom, W_i_bond,
           W_h0, W_h1, W_lr, gru_bias, Wih_f, Whh_f, bih_f, bhh_f,
           Wih_b, Whh_b, bih_b, bhh_b, W_o, b_o):
    del a_scope
    f_atoms_p = jnp.pad(f_atoms, ((0, A_PAD - A), (0, 0)))
    a2b_flat = jnp.pad(a2b, ((0, A_PAD - A), (0, 0))).reshape(-1)

    input_atom = _mm(f_atoms_p, W_i_atom.T, 1024, True)
    input_bond = _mm(f_bonds, W_i_bond.T, 1280, True)

    msg_atom = input_atom
    mb = input_bond
    for W_h in (W_h0, W_h1):
        agg = _sc_agg(mb, a2b_flat)
        msg_atom = _add(msg_atom, agg)
        pre = _sc_pre(msg_atom, mb, b2a, b2revb)
        mb = _bond_mm(pre, input_bond, W_h.T)

    agg3 = _sc_agg(mb, a2b_flat)
    w_lr_t = W_lr.T
    hidden, message = _mm3(agg3, msg_atom, input_atom,
                           w_lr_t[:H], w_lr_t[H:2 * H], w_lr_t[2 * H:],
                           gru_bias.reshape(1, H))

    hid3 = hidden[1:A].reshape(NM, L, H)
    h0 = _h0_kernel(hid3)
    h0p = jnp.pad(h0, ((0, NMP - NM), (0, 0)))
    seq = jnp.transpose(message[1:A].reshape(NM, L, H), (1, 0, 2))
    seq = jnp.pad(seq, ((0, 0), (0, NMP - NM), (0, 0))).reshape(L * NMP, H)

    wot = W_o.T
    out_main, row0 = _gru_kernel(
        seq, h0p, message[0:8],
        Wih_f.T, Whh_f.T, bih_f.reshape(1, -1), bhh_f.reshape(1, -1),
        Wih_b.T, Whh_b.T, bih_b.reshape(1, -1), bhh_b.reshape(1, -1),
        wot[:H], wot[H:], b_o.reshape(1, H))

    og = out_main.reshape(L, NMP, H)[:, :NM, :]
    og = jnp.transpose(og, (1, 0, 2)).reshape(NM * L, H)
    return jnp.concatenate([row0[0:1], og], axis=0)


# speedup vs baseline: 1.4099x; 1.4099x over previous
import functools

import jax
import jax.numpy as jnp
from jax import lax
from jax.experimental import pallas as pl
from jax.experimental.pallas import tpu as pltpu
from jax.experimental.pallas import tpu_sc as plsc

A = 10001
E = 320000
MAXB = 32
H = 128
NM = 100
L = 100
A_PAD = 10240
NMP = 128

_NW = 32
_APW = A_PAD // _NW
_GA = 4
_NGA = _APW // _GA
_EPW = E // _NW
_GB = 128
_NGB = _EPW // _GB
_GBR = _EPW - _NGB * _GB
_NBUF = 4


def _mm(x, wt, block_rows, relu):
    n, k = x.shape
    h = wt.shape[1]

    def body(x_ref, w_ref, o_ref):
        y = jnp.dot(x_ref[...], w_ref[...], preferred_element_type=jnp.float32)
        if relu:
            y = jnp.maximum(y, 0.0)
        o_ref[...] = y

    return pl.pallas_call(
        body,
        grid=(n // block_rows,),
        in_specs=[pl.BlockSpec((block_rows, k), lambda i: (i, 0)),
                  pl.BlockSpec((k, h), lambda i: (0, 0))],
        out_specs=pl.BlockSpec((block_rows, h), lambda i: (i, 0)),
        out_shape=jax.ShapeDtypeStruct((n, h), jnp.float32),
    )(x, wt)


def _add(msg, agg):
    blk = 1024

    def body(m_ref, a_ref, o_ref):
        o_ref[...] = m_ref[...] + a_ref[...]

    return pl.pallas_call(
        body,
        grid=(A_PAD // blk,),
        in_specs=[pl.BlockSpec((blk, H), lambda i: (i, 0)),
                  pl.BlockSpec((blk, H), lambda i: (i, 0))],
        out_specs=pl.BlockSpec((blk, H), lambda i: (i, 0)),
        out_shape=jax.ShapeDtypeStruct((A_PAD, H), jnp.float32),
    )(msg, agg)


def _bond_mm(pre, ib, wt):
    blk = 1280

    def body(p_ref, i_ref, w_ref, o_ref):
        o_ref[...] = jnp.maximum(
            i_ref[...] + jnp.dot(p_ref[...], w_ref[...],
                                 preferred_element_type=jnp.float32), 0.0)

    return pl.pallas_call(
        body,
        grid=(E // blk,),
        in_specs=[pl.BlockSpec((blk, H), lambda i: (i, 0)),
                  pl.BlockSpec((blk, H), lambda i: (i, 0)),
                  pl.BlockSpec((H, H), lambda i: (0, 0))],
        out_specs=pl.BlockSpec((blk, H), lambda i: (i, 0)),
        out_shape=jax.ShapeDtypeStruct((E, H), jnp.float32),
    )(pre, ib, wt)


def _mm3(agg3, msg2, ia, w0, w1, w2, bias):
    blk = 1024

    def body(a_ref, m_ref, i_ref, w0_ref, w1_ref, w2_ref, b_ref, hid_ref, msg_ref):
        h = (jnp.dot(a_ref[...], w0_ref[...], preferred_element_type=jnp.float32)
             + jnp.dot(m_ref[...], w1_ref[...], preferred_element_type=jnp.float32)
             + jnp.dot(i_ref[...], w2_ref[...], preferred_element_type=jnp.float32))
        hid_ref[...] = h
        msg_ref[...] = jnp.maximum(h + b_ref[...], 0.0)

    return pl.pallas_call(
        body,
        grid=(A_PAD // blk,),
        in_specs=[pl.BlockSpec((blk, H), lambda i: (i, 0)),
                  pl.BlockSpec((blk, H), lambda i: (i, 0)),
                  pl.BlockSpec((blk, H), lambda i: (i, 0)),
                  pl.BlockSpec((H, H), lambda i: (0, 0)),
                  pl.BlockSpec((H, H), lambda i: (0, 0)),
                  pl.BlockSpec((H, H), lambda i: (0, 0)),
                  pl.BlockSpec((1, H), lambda i: (0, 0))],
        out_specs=[pl.BlockSpec((blk, H), lambda i: (i, 0)),
                   pl.BlockSpec((blk, H), lambda i: (i, 0))],
        out_shape=[jax.ShapeDtypeStruct((A_PAD, H), jnp.float32),
                   jax.ShapeDtypeStruct((A_PAD, H), jnp.float32)],
    )(agg3, msg2, ia, w0, w1, w2, bias)


def _h0_kernel(hid3):
    def body(h_ref, o_ref):
        o_ref[...] = jnp.max(h_ref[...], axis=1)

    return pl.pallas_call(
        body,
        out_shape=jax.ShapeDtypeStruct((NM, H), jnp.float32),
    )(hid3)


def _gru_kernel(seq, h0p, m0, wif, whf, bif, bhf, wib, whb, bib, bhb, wo1, wo2, bo):
    def body(seq_ref, h0_ref, m0_ref, wif_ref, whf_ref, bif_ref, bhf_ref,
             wib_ref, whb_ref, bib_ref, bhb_ref, wo1_ref, wo2_ref, bo_ref,
             out_ref, r0_ref):
        out_ref[...] = jnp.zeros_like(out_ref)
        wifv = wif_ref[...]
        whfv = whf_ref[...]
        wibv = wib_ref[...]
        whbv = whb_ref[...]
        bifv = bif_ref[...]
        bhfv = bhf_ref[...]
        bibv = bib_ref[...]
        bhbv = bhb_ref[...]
        wo1v = wo1_ref[...]
        wo2v = wo2_ref[...]

        def step(h, x, wih, whh, bi, bh):
            gi = jnp.dot(x, wih, preferred_element_type=jnp.float32) + bi
            gh = jnp.dot(h, whh, preferred_element_type=jnp.float32) + bh
            r = jax.nn.sigmoid(gi[:, :H] + gh[:, :H])
            z = jax.nn.sigmoid(gi[:, H:2 * H] + gh[:, H:2 * H])
            n = jnp.tanh(gi[:, 2 * H:] + r * gh[:, 2 * H:])
            return (1.0 - z) * n + z * h

        def loop(t, carry):
            hf, hb = carry
            xf = seq_ref[pl.ds(t * NMP, NMP), :]
            xb = seq_ref[pl.ds((L - 1 - t) * NMP, NMP), :]
            hf = step(hf, xf, wifv, whfv, bifv, bhfv)
            hb = step(hb, xb, wibv, whbv, bibv, bhbv)
            out_ref[pl.ds(t * NMP, NMP), :] += jnp.dot(
                hf, wo1v, preferred_element_type=jnp.float32)
            out_ref[pl.ds((L - 1 - t) * NMP, NMP), :] += jnp.dot(
                hb, wo2v, preferred_element_type=jnp.float32)
            return (hf, hb)

        h0 = h0_ref[...]
        lax.fori_loop(0, L, loop, (h0, h0))
        out_ref[...] = jnp.maximum(out_ref[...] + bo_ref[...], 0.0)
        r0_ref[...] = jnp.maximum(
            jnp.dot(m0_ref[...], wo1v + wo2v, preferred_element_type=jnp.float32)
            + bo_ref[...], 0.0)

    return pl.pallas_call(
        body,
        out_shape=[jax.ShapeDtypeStruct((L * NMP, H), jnp.float32),
                   jax.ShapeDtypeStruct((8, H), jnp.float32)],
    )(seq, h0p, m0, wif, whf, bif, bhf, wib, whb, bib, bhb, wo1, wo2, bo)


def _sc_mesh():
    return plsc.VectorSubcoreMesh(core_axis_name="c", subcore_axis_name="s")


def _sc_agg(mb, a2b_flat):
    @functools.partial(
        pl.kernel,
        mesh=_sc_mesh(),
        out_type=jax.ShapeDtypeStruct((A_PAD, H), jnp.float32),
        scratch_types=[
            pltpu.VMEM((_APW * MAXB,), jnp.int32),
            pltpu.VMEM((_NBUF, _GA * MAXB, H), jnp.float32),
            pltpu.VMEM((_APW, H), jnp.float32),
            pltpu.SemaphoreType.DMA,
        ],
    )
    def k(mb_ref, idx_ref, out_ref, idx_v, rows_v, out_v, sem):
        wid = lax.axis_index("s") * 2 + lax.axis_index("c")
        pltpu.sync_copy(idx_ref.at[pl.ds(wid * (_APW * MAXB), _APW * MAXB)], idx_v)

        def batch(k_, carry):
            g0 = k_ * _NBUF
            handles = [
                pltpu.async_copy(
                    mb_ref.at[idx_v.at[pl.ds((g0 + b) * (_GA * MAXB), _GA * MAXB)]],
                    rows_v.at[b], sem)
                for b in range(_NBUF)
            ]
            for h in handles:
                h.wait()
            for b in range(_NBUF):
                def atom(i, c2, b=b):
                    base = i * MAXB
                    for c in range(H // 16):
                        sl = pl.ds(c * 16, 16)
                        v = rows_v[b, base, sl]
                        s = v
                        m = v
                        for j in range(1, MAXB):
                            v = rows_v[b, base + j, sl]
                            s = s + v
                            m = jnp.maximum(m, v)
                        out_v[(g0 + b) * _GA + i, sl] = s * m
                    return c2

                lax.fori_loop(0, _GA, atom, 0)
            return carry

        lax.fori_loop(0, _NGA // _NBUF, batch, 0)
        pltpu.sync_copy(out_v, out_ref.at[pl.ds(wid * _APW, _APW)])

    return k(mb, a2b_flat)


def _sc_pre(t_tab, u_tab, b2a, b2revb):
    @functools.partial(
        pl.kernel,
        mesh=_sc_mesh(),
        out_type=jax.ShapeDtypeStruct((E, H), jnp.float32),
        scratch_types=[
            pltpu.VMEM((_EPW,), jnp.int32),
            pltpu.VMEM((_EPW,), jnp.int32),
            pltpu.VMEM((2, _GB, H), jnp.float32),
            pltpu.VMEM((2, _GB, H), jnp.float32),
            pltpu.VMEM((2, _GB, H), jnp.float32),
            pltpu.SemaphoreType.DMA,
        ],
    )
    def k(t_ref, u_ref, ba_ref, br_ref, out_ref,
          ia_v, ir_v, t_v, u_v, o_v, sem):
        wid = lax.axis_index("s") * 2 + lax.axis_index("c")
        base = wid * _EPW
        pltpu.sync_copy(ba_ref.at[pl.ds(base, _EPW)], ia_v)
        pltpu.sync_copy(br_ref.at[pl.ds(base, _EPW)], ir_v)

        def compute_rows(s_, n_rows):
            def row(r, c2):
                for cc in range(H // 16):
                    sl = pl.ds(cc * 16, 16)
                    o_v[s_, r, sl] = t_v[s_, r, sl] - u_v[s_, r, sl]
                return c2

            lax.fori_loop(0, n_rows, row, 0)

        def batch(k_, carry):
            g0 = k_ * 2
            handles = []
            for s_ in range(2):
                g = g0 + s_
                handles.append(pltpu.async_copy(
                    t_ref.at[ia_v.at[pl.ds(g * _GB, _GB)]], t_v.at[s_], sem))
                handles.append(pltpu.async_copy(
                    u_ref.at[ir_v.at[pl.ds(g * _GB, _GB)]], u_v.at[s_], sem))
            for h in handles:
                h.wait()
            for s_ in range(2):
                compute_rows(s_, _GB)
                pltpu.sync_copy(o_v.at[s_],
                                out_ref.at[pl.ds(base + (g0 + s_) * _GB, _GB)])
            return carry

        lax.fori_loop(0, _NGB // 2, batch, 0)

        roff = _NGB * _GB
        ct = pltpu.async_copy(
            t_ref.at[ia_v.at[pl.ds(roff, _GBR)]], t_v.at[0, pl.ds(0, _GBR)], sem)
        cu = pltpu.async_copy(
            u_ref.at[ir_v.at[pl.ds(roff, _GBR)]], u_v.at[0, pl.ds(0, _GBR)], sem)
        ct.wait()
        cu.wait()
        compute_rows(0, _GBR)
        pltpu.sync_copy(o_v.at[0, pl.ds(0, _GBR)],
                        out_ref.at[pl.ds(base + roff, _GBR)])

    return k(t_tab, u_tab, b2a, b2revb)


def kernel(f_atoms, f_bonds, a2b, b2a, b2revb, a_scope, W_i_atom, W_i_bond,
           W_h0, W_h1, W_lr, gru_bias, Wih_f, Whh_f, bih_f, bhh_f,
           Wih_b, Whh_b, bih_b, bhh_b, W_o, b_o):
    del a_scope
    f_atoms_p = jnp.pad(f_atoms, ((0, A_PAD - A), (0, 0)))
    pad_idx = (jnp.arange((A_PAD - A) * MAXB, dtype=jnp.int32) * 997) % E
    a2b_flat = jnp.concatenate([a2b.reshape(-1), pad_idx])

    input_atom = _mm(f_atoms_p, W_i_atom.T, 1024, True)
    input_bond = _mm(f_bonds, W_i_bond.T, 1280, True)

    msg_atom = input_atom
    mb = input_bond
    for W_h in (W_h0, W_h1):
        agg = _sc_agg(mb, a2b_flat)
        msg_atom = _add(msg_atom, agg)
        pre = _sc_pre(msg_atom, mb, b2a, b2revb)
        mb = _bond_mm(pre, input_bond, W_h.T)

    agg3 = _sc_agg(mb, a2b_flat)
    w_lr_t = W_lr.T
    hidden, message = _mm3(agg3, msg_atom, input_atom,
                           w_lr_t[:H], w_lr_t[H:2 * H], w_lr_t[2 * H:],
                           gru_bias.reshape(1, H))

    hid3 = hidden[1:A].reshape(NM, L, H)
    h0 = _h0_kernel(hid3)
    h0p = jnp.pad(h0, ((0, NMP - NM), (0, 0)))
    seq = jnp.transpose(message[1:A].reshape(NM, L, H), (1, 0, 2))
    seq = jnp.pad(seq, ((0, 0), (0, NMP - NM), (0, 0))).reshape(L * NMP, H)

    wot = W_o.T
    out_main, row0 = _gru_kernel(
        seq, h0p, message[0:8],
        Wih_f.T, Whh_f.T, bih_f.reshape(1, -1), bhh_f.reshape(1, -1),
        Wih_b.T, Whh_b.T, bih_b.reshape(1, -1), bhh_b.reshape(1, -1),
        wot[:H], wot[H:], b_o.reshape(1, H))

    og = out_main.reshape(L, NMP, H)[:, :NM, :]
    og = jnp.transpose(og, (1, 0, 2)).reshape(NM * L, H)
    return jnp.concatenate([row0[0:1], og], axis=0)


# speedup vs baseline: 1.4109x; 1.0008x over previous
import functools

import jax
import jax.numpy as jnp
from jax import lax
from jax.experimental import pallas as pl
from jax.experimental.pallas import tpu as pltpu
from jax.experimental.pallas import tpu_sc as plsc

A = 10001
E = 320000
MAXB = 32
H = 128
NM = 100
L = 100
A_PAD = 10240
NMP = 128

_NW = 32
_APW = A_PAD // _NW
_GA = 4
_NGA = _APW // _GA
_EPW = E // _NW
_GB = 128
_NGB = _EPW // _GB
_GBR = _EPW - _NGB * _GB
_NBUF = 4


def _mm(x, wt, block_rows, relu):
    n, k = x.shape
    h = wt.shape[1]

    def body(x_ref, w_ref, o_ref):
        y = jnp.dot(x_ref[...], w_ref[...], preferred_element_type=jnp.float32)
        if relu:
            y = jnp.maximum(y, 0.0)
        o_ref[...] = y

    return pl.pallas_call(
        body,
        grid=(n // block_rows,),
        in_specs=[pl.BlockSpec((block_rows, k), lambda i: (i, 0)),
                  pl.BlockSpec((k, h), lambda i: (0, 0))],
        out_specs=pl.BlockSpec((block_rows, h), lambda i: (i, 0)),
        out_shape=jax.ShapeDtypeStruct((n, h), jnp.float32),
    )(x, wt)


def _add(msg, agg):
    blk = 1024

    def body(m_ref, a_ref, o_ref):
        o_ref[...] = m_ref[...] + a_ref[...]

    return pl.pallas_call(
        body,
        grid=(A_PAD // blk,),
        in_specs=[pl.BlockSpec((blk, H), lambda i: (i, 0)),
                  pl.BlockSpec((blk, H), lambda i: (i, 0))],
        out_specs=pl.BlockSpec((blk, H), lambda i: (i, 0)),
        out_shape=jax.ShapeDtypeStruct((A_PAD, H), jnp.float32),
    )(msg, agg)


def _bond_mm(pre, ib, wt):
    blk = 1280

    def body(p_ref, i_ref, w_ref, o_ref):
        o_ref[...] = jnp.maximum(
            i_ref[...] + jnp.dot(p_ref[...], w_ref[...],
                                 preferred_element_type=jnp.float32), 0.0)

    return pl.pallas_call(
        body,
        grid=(E // blk,),
        in_specs=[pl.BlockSpec((blk, H), lambda i: (i, 0)),
                  pl.BlockSpec((blk, H), lambda i: (i, 0)),
                  pl.BlockSpec((H, H), lambda i: (0, 0))],
        out_specs=pl.BlockSpec((blk, H), lambda i: (i, 0)),
        out_shape=jax.ShapeDtypeStruct((E, H), jnp.float32),
    )(pre, ib, wt)


def _mm3(agg3, msg2, ia, w0, w1, w2, bias):
    blk = 1024

    def body(a_ref, m_ref, i_ref, w0_ref, w1_ref, w2_ref, b_ref, hid_ref, msg_ref):
        h = (jnp.dot(a_ref[...], w0_ref[...], preferred_element_type=jnp.float32)
             + jnp.dot(m_ref[...], w1_ref[...], preferred_element_type=jnp.float32)
             + jnp.dot(i_ref[...], w2_ref[...], preferred_element_type=jnp.float32))
        hid_ref[...] = h
        msg_ref[...] = jnp.maximum(h + b_ref[...], 0.0)

    return pl.pallas_call(
        body,
        grid=(A_PAD // blk,),
        in_specs=[pl.BlockSpec((blk, H), lambda i: (i, 0)),
                  pl.BlockSpec((blk, H), lambda i: (i, 0)),
                  pl.BlockSpec((blk, H), lambda i: (i, 0)),
                  pl.BlockSpec((H, H), lambda i: (0, 0)),
                  pl.BlockSpec((H, H), lambda i: (0, 0)),
                  pl.BlockSpec((H, H), lambda i: (0, 0)),
                  pl.BlockSpec((1, H), lambda i: (0, 0))],
        out_specs=[pl.BlockSpec((blk, H), lambda i: (i, 0)),
                   pl.BlockSpec((blk, H), lambda i: (i, 0))],
        out_shape=[jax.ShapeDtypeStruct((A_PAD, H), jnp.float32),
                   jax.ShapeDtypeStruct((A_PAD, H), jnp.float32)],
    )(agg3, msg2, ia, w0, w1, w2, bias)


_SPW = (L * NMP) // _NW
_SGB = 80
_NSG = _SPW // _SGB
_MPW = 4


def _sc_seq(message, hidden, tidx):
    @functools.partial(
        pl.kernel,
        mesh=_sc_mesh(),
        out_type=[jax.ShapeDtypeStruct((L * NMP, H), jnp.float32),
                  jax.ShapeDtypeStruct((NMP, H), jnp.float32)],
        scratch_types=[
            pltpu.VMEM((_SPW,), jnp.int32),
            pltpu.VMEM((_SPW, H), jnp.float32),
            pltpu.VMEM((_MPW * L + 8, H), jnp.float32),
            pltpu.VMEM((_MPW, H), jnp.float32),
            pltpu.SemaphoreType.DMA,
        ],
    )
    def k(msg_ref, hid_ref, tidx_ref, seq_ref, h0_ref,
          idx_v, rows_v, hrows_v, h0_v, sem):
        wid = lax.axis_index("s") * 2 + lax.axis_index("c")
        base = wid * _SPW
        pltpu.sync_copy(tidx_ref.at[pl.ds(base, _SPW)], idx_v)
        handles = [
            pltpu.async_copy(
                msg_ref.at[idx_v.at[pl.ds(g * _SGB, _SGB)]],
                rows_v.at[pl.ds(g * _SGB, _SGB)], sem)
            for g in range(_NSG)
        ]

        @pl.when(wid < NM // _MPW)
        def _():
            pltpu.sync_copy(hid_ref.at[pl.ds(wid * (_MPW * L), _MPW * L + 8)],
                            hrows_v)
            for mi in range(_MPW):
                for c in range(H // 16):
                    sl = pl.ds(c * 16, 16)

                    def red(j, m, mi=mi, sl=sl):
                        return jnp.maximum(m, hrows_v[1 + mi * L + j, sl])

                    h0_v[mi, sl] = lax.fori_loop(
                        1, L, red, hrows_v[1 + mi * L, sl])
            pltpu.sync_copy(h0_v, h0_ref.at[pl.ds(wid * _MPW, _MPW)])

        @pl.when((wid >= NM // _MPW) & (wid < NM // _MPW + (NMP - NM) // _MPW))
        def _():
            for mi in range(_MPW):
                for c in range(H // 16):
                    h0_v[mi, pl.ds(c * 16, 16)] = jnp.zeros((16,), jnp.float32)
            pltpu.sync_copy(
                h0_v, h0_ref.at[pl.ds(NM + (wid - NM // _MPW) * _MPW, _MPW)])

        for h in handles:
            h.wait()
        pltpu.sync_copy(rows_v, seq_ref.at[pl.ds(base, _SPW)])

    return k(message, hidden, tidx)


def _gru_kernel(seq, h0p, m0, wif, whf, bif, bhf, wib, whb, bib, bhb, wo1, wo2, bo):
    def body(seq_ref, h0_ref, m0_ref, wif_ref, whf_ref, bif_ref, bhf_ref,
             wib_ref, whb_ref, bib_ref, bhb_ref, wo1_ref, wo2_ref, bo_ref,
             out_ref, r0_ref):
        out_ref[...] = jnp.zeros_like(out_ref)
        wifv = wif_ref[...]
        whfv = whf_ref[...]
        wibv = wib_ref[...]
        whbv = whb_ref[...]
        bifv = bif_ref[...]
        bhfv = bhf_ref[...]
        bibv = bib_ref[...]
        bhbv = bhb_ref[...]
        wo1v = wo1_ref[...]
        wo2v = wo2_ref[...]

        def step(h, x, wih, whh, bi, bh):
            gi = jnp.dot(x, wih, preferred_element_type=jnp.float32) + bi
            gh = jnp.dot(h, whh, preferred_element_type=jnp.float32) + bh
            r = jax.nn.sigmoid(gi[:, :H] + gh[:, :H])
            z = jax.nn.sigmoid(gi[:, H:2 * H] + gh[:, H:2 * H])
            n = jnp.tanh(gi[:, 2 * H:] + r * gh[:, 2 * H:])
            return (1.0 - z) * n + z * h

        def loop(t, carry):
            hf, hb = carry
            xf = seq_ref[pl.ds(t * NMP, NMP), :]
            xb = seq_ref[pl.ds((L - 1 - t) * NMP, NMP), :]
            hf = step(hf, xf, wifv, whfv, bifv, bhfv)
            hb = step(hb, xb, wibv, whbv, bibv, bhbv)
            out_ref[pl.ds(t * NMP, NMP), :] += jnp.dot(
                hf, wo1v, preferred_element_type=jnp.float32)
            out_ref[pl.ds((L - 1 - t) * NMP, NMP), :] += jnp.dot(
                hb, wo2v, preferred_element_type=jnp.float32)
            return (hf, hb)

        h0 = h0_ref[...]
        lax.fori_loop(0, L, loop, (h0, h0))
        out_ref[...] = jnp.maximum(out_ref[...] + bo_ref[...], 0.0)
        r0_ref[...] = jnp.maximum(
            jnp.dot(m0_ref[...], wo1v + wo2v, preferred_element_type=jnp.float32)
            + bo_ref[...], 0.0)

    return pl.pallas_call(
        body,
        out_shape=[jax.ShapeDtypeStruct((L * NMP, H), jnp.float32),
                   jax.ShapeDtypeStruct((8, H), jnp.float32)],
    )(seq, h0p, m0, wif, whf, bif, bhf, wib, whb, bib, bhb, wo1, wo2, bo)


def _sc_mesh():
    return plsc.VectorSubcoreMesh(core_axis_name="c", subcore_axis_name="s")


def _sc_agg(mb, a2b_flat):
    @functools.partial(
        pl.kernel,
        mesh=_sc_mesh(),
        out_type=jax.ShapeDtypeStruct((A_PAD, H), jnp.float32),
        scratch_types=[
            pltpu.VMEM((_APW * MAXB,), jnp.int32),
            pltpu.VMEM((_NBUF, _GA * MAXB, H), jnp.float32),
            pltpu.VMEM((_APW, H), jnp.float32),
            pltpu.SemaphoreType.DMA,
        ],
    )
    def k(mb_ref, idx_ref, out_ref, idx_v, rows_v, out_v, sem):
        wid = lax.axis_index("s") * 2 + lax.axis_index("c")
        pltpu.sync_copy(idx_ref.at[pl.ds(wid * (_APW * MAXB), _APW * MAXB)], idx_v)

        def batch(k_, carry):
            g0 = k_ * _NBUF
            handles = [
                pltpu.async_copy(
                    mb_ref.at[idx_v.at[pl.ds((g0 + b) * (_GA * MAXB), _GA * MAXB)]],
                    rows_v.at[b], sem)
                for b in range(_NBUF)
            ]
            for h in handles:
                h.wait()
            for b in range(_NBUF):
                def atom(i, c2, b=b):
                    base = i * MAXB
                    for c in range(H // 16):
                        sl = pl.ds(c * 16, 16)
                        v = rows_v[b, base, sl]
                        s = v
                        m = v
                        for j in range(1, MAXB):
                            v = rows_v[b, base + j, sl]
                            s = s + v
                            m = jnp.maximum(m, v)
                        out_v[(g0 + b) * _GA + i, sl] = s * m
                    return c2

                lax.fori_loop(0, _GA, atom, 0)
            return carry

        lax.fori_loop(0, _NGA // _NBUF, batch, 0)
        pltpu.sync_copy(out_v, out_ref.at[pl.ds(wid * _APW, _APW)])

    return k(mb, a2b_flat)


def _sc_pre(t_tab, u_tab, b2a, b2revb):
    @functools.partial(
        pl.kernel,
        mesh=_sc_mesh(),
        out_type=jax.ShapeDtypeStruct((E, H), jnp.float32),
        scratch_types=[
            pltpu.VMEM((_EPW,), jnp.int32),
            pltpu.VMEM((_EPW,), jnp.int32),
            pltpu.VMEM((2, _GB, H), jnp.float32),
            pltpu.VMEM((2, _GB, H), jnp.float32),
            pltpu.VMEM((2, _GB, H), jnp.float32),
            pltpu.SemaphoreType.DMA,
        ],
    )
    def k(t_ref, u_ref, ba_ref, br_ref, out_ref,
          ia_v, ir_v, t_v, u_v, o_v, sem):
        wid = lax.axis_index("s") * 2 + lax.axis_index("c")
        base = wid * _EPW
        pltpu.sync_copy(ba_ref.at[pl.ds(base, _EPW)], ia_v)
        pltpu.sync_copy(br_ref.at[pl.ds(base, _EPW)], ir_v)

        def compute_rows(s_, n_rows):
            def row(r, c2):
                for cc in range(H // 16):
                    sl = pl.ds(cc * 16, 16)
                    o_v[s_, r, sl] = t_v[s_, r, sl] - u_v[s_, r, sl]
                return c2

            lax.fori_loop(0, n_rows, row, 0)

        def batch(k_, carry):
            g0 = k_ * 2
            handles = []
            for s_ in range(2):
                g = g0 + s_
                handles.append(pltpu.async_copy(
                    t_ref.at[ia_v.at[pl.ds(g * _GB, _GB)]], t_v.at[s_], sem))
                handles.append(pltpu.async_copy(
                    u_ref.at[ir_v.at[pl.ds(g * _GB, _GB)]], u_v.at[s_], sem))
            for h in handles:
                h.wait()
            for s_ in range(2):
                compute_rows(s_, _GB)
                pltpu.sync_copy(o_v.at[s_],
                                out_ref.at[pl.ds(base + (g0 + s_) * _GB, _GB)])
            return carry

        lax.fori_loop(0, _NGB // 2, batch, 0)

        roff = _NGB * _GB
        ct = pltpu.async_copy(
            t_ref.at[ia_v.at[pl.ds(roff, _GBR)]], t_v.at[0, pl.ds(0, _GBR)], sem)
        cu = pltpu.async_copy(
            u_ref.at[ir_v.at[pl.ds(roff, _GBR)]], u_v.at[0, pl.ds(0, _GBR)], sem)
        ct.wait()
        cu.wait()
        compute_rows(0, _GBR)
        pltpu.sync_copy(o_v.at[0, pl.ds(0, _GBR)],
                        out_ref.at[pl.ds(base + roff, _GBR)])

    return k(t_tab, u_tab, b2a, b2revb)


def kernel(f_atoms, f_bonds, a2b, b2a, b2revb, a_scope, W_i_atom, W_i_bond,
           W_h0, W_h1, W_lr, gru_bias, Wih_f, Whh_f, bih_f, bhh_f,
           Wih_b, Whh_b, bih_b, bhh_b, W_o, b_o):
    del a_scope
    f_atoms_p = jnp.pad(f_atoms, ((0, A_PAD - A), (0, 0)))
    pad_idx = (jnp.arange((A_PAD - A) * MAXB, dtype=jnp.int32) * 997) % E
    a2b_flat = jnp.concatenate([a2b.reshape(-1), pad_idx])

    input_atom = _mm(f_atoms_p, W_i_atom.T, 1024, True)
    input_bond = _mm(f_bonds, W_i_bond.T, 1280, True)

    msg_atom = input_atom
    mb = input_bond
    for W_h in (W_h0, W_h1):
        agg = _sc_agg(mb, a2b_flat)
        msg_atom = _add(msg_atom, agg)
        pre = _sc_pre(msg_atom, mb, b2a, b2revb)
        mb = _bond_mm(pre, input_bond, W_h.T)

    agg3 = _sc_agg(mb, a2b_flat)
    w_lr_t = W_lr.T
    hidden, message = _mm3(agg3, msg_atom, input_atom,
                           w_lr_t[:H], w_lr_t[H:2 * H], w_lr_t[2 * H:],
                           gru_bias.reshape(1, H))

    tt = jnp.arange(L * NMP, dtype=jnp.int32)
    tidx = 1 + (tt % NMP % NM) * L + tt // NMP
    seq, h0p = _sc_seq(message, hidden, tidx)

    wot = W_o.T
    out_main, row0 = _gru_kernel(
        seq, h0p, message[0:8],
        Wih_f.T, Whh_f.T, bih_f.reshape(1, -1), bhh_f.reshape(1, -1),
        Wih_b.T, Whh_b.T, bih_b.reshape(1, -1), bhh_b.reshape(1, -1),
        wot[:H], wot[H:], b_o.reshape(1, H))

    og = out_main.reshape(L, NMP, H)[:, :NM, :]
    og = jnp.transpose(og, (1, 0, 2)).reshape(NM * L, H)
    return jnp.concatenate([row0[0:1], og], axis=0)


# speedup vs baseline: 1.5736x; 1.1153x over previous
import functools

import jax
import jax.numpy as jnp
from jax import lax
from jax.experimental import pallas as pl
from jax.experimental.pallas import tpu as pltpu
from jax.experimental.pallas import tpu_sc as plsc

A = 10001
E = 320000
MAXB = 32
H = 128
NM = 100
L = 100
A_PAD = 10240
NMP = 128

_NW = 32
_APW = A_PAD // _NW
_GA = 4
_NGA = _APW // _GA
_EPW = E // _NW
_GB = 128
_NGB = _EPW // _GB
_GBR = _EPW - _NGB * _GB
_NBUF = 4


def _mm(x, wt, block_rows, relu):
    n, k = x.shape
    h = wt.shape[1]

    def body(x_ref, w_ref, o_ref):
        y = jnp.dot(x_ref[...], w_ref[...], preferred_element_type=jnp.float32)
        if relu:
            y = jnp.maximum(y, 0.0)
        o_ref[...] = y

    return pl.pallas_call(
        body,
        grid=(n // block_rows,),
        in_specs=[pl.BlockSpec((block_rows, k), lambda i: (i, 0)),
                  pl.BlockSpec((k, h), lambda i: (0, 0))],
        out_specs=pl.BlockSpec((block_rows, h), lambda i: (i, 0)),
        out_shape=jax.ShapeDtypeStruct((n, h), jnp.float32),
    )(x, wt)


def _mm_t(xt, wt, block_cols, relu):
    k, n = xt.shape
    h = wt.shape[1]

    def body(x_ref, w_ref, o_ref):
        y = lax.dot_general(x_ref[...], w_ref[...],
                            (((0,), (0,)), ((), ())),
                            preferred_element_type=jnp.float32)
        if relu:
            y = jnp.maximum(y, 0.0)
        o_ref[...] = y

    return pl.pallas_call(
        body,
        grid=(n // block_cols,),
        in_specs=[pl.BlockSpec((k, block_cols), lambda i: (0, i)),
                  pl.BlockSpec((k, h), lambda i: (0, 0))],
        out_specs=pl.BlockSpec((block_cols, h), lambda i: (i, 0)),
        out_shape=jax.ShapeDtypeStruct((n, h), jnp.float32),
    )(xt, wt)


def _add(msg, agg):
    blk = 1024

    def body(m_ref, a_ref, o_ref):
        o_ref[...] = m_ref[...] + a_ref[...]

    return pl.pallas_call(
        body,
        grid=(A_PAD // blk,),
        in_specs=[pl.BlockSpec((blk, H), lambda i: (i, 0)),
                  pl.BlockSpec((blk, H), lambda i: (i, 0))],
        out_specs=pl.BlockSpec((blk, H), lambda i: (i, 0)),
        out_shape=jax.ShapeDtypeStruct((A_PAD, H), jnp.float32),
    )(msg, agg)


def _bond_mm(pre, ib, wt):
    blk = 1280

    def body(p_ref, i_ref, w_ref, o_ref):
        o_ref[...] = jnp.maximum(
            i_ref[...] + jnp.dot(p_ref[...], w_ref[...],
                                 preferred_element_type=jnp.float32), 0.0)

    return pl.pallas_call(
        body,
        grid=(E // blk,),
        in_specs=[pl.BlockSpec((blk, H), lambda i: (i, 0)),
                  pl.BlockSpec((blk, H), lambda i: (i, 0)),
                  pl.BlockSpec((H, H), lambda i: (0, 0))],
        out_specs=pl.BlockSpec((blk, H), lambda i: (i, 0)),
        out_shape=jax.ShapeDtypeStruct((E, H), jnp.float32),
    )(pre, ib, wt)


def _mm3(agg3, msg2, ia, w0, w1, w2, bias):
    blk = 1024

    def body(a_ref, m_ref, i_ref, w0_ref, w1_ref, w2_ref, b_ref, hid_ref, msg_ref):
        h = (jnp.dot(a_ref[...], w0_ref[...], preferred_element_type=jnp.float32)
             + jnp.dot(m_ref[...], w1_ref[...], preferred_element_type=jnp.float32)
             + jnp.dot(i_ref[...], w2_ref[...], preferred_element_type=jnp.float32))
        hid_ref[...] = h
        msg_ref[...] = jnp.maximum(h + b_ref[...], 0.0)

    return pl.pallas_call(
        body,
        grid=(A_PAD // blk,),
        in_specs=[pl.BlockSpec((blk, H), lambda i: (i, 0)),
                  pl.BlockSpec((blk, H), lambda i: (i, 0)),
                  pl.BlockSpec((blk, H), lambda i: (i, 0)),
                  pl.BlockSpec((H, H), lambda i: (0, 0)),
                  pl.BlockSpec((H, H), lambda i: (0, 0)),
                  pl.BlockSpec((H, H), lambda i: (0, 0)),
                  pl.BlockSpec((1, H), lambda i: (0, 0))],
        out_specs=[pl.BlockSpec((blk, H), lambda i: (i, 0)),
                   pl.BlockSpec((blk, H), lambda i: (i, 0))],
        out_shape=[jax.ShapeDtypeStruct((A_PAD, H), jnp.float32),
                   jax.ShapeDtypeStruct((A_PAD, H), jnp.float32)],
    )(agg3, msg2, ia, w0, w1, w2, bias)


_SPW = (L * NMP) // _NW
_SGB = 80
_NSG = _SPW // _SGB
_MPW = 4


def _sc_seq(message, hidden, tidx):
    @functools.partial(
        pl.kernel,
        mesh=_sc_mesh(),
        out_type=[jax.ShapeDtypeStruct((L * NMP, H), jnp.float32),
                  jax.ShapeDtypeStruct((NMP, H), jnp.float32)],
        scratch_types=[
            pltpu.VMEM((_SPW,), jnp.int32),
            pltpu.VMEM((_SPW, H), jnp.float32),
            pltpu.VMEM((_MPW * L + 8, H), jnp.float32),
            pltpu.VMEM((_MPW, H), jnp.float32),
            pltpu.SemaphoreType.DMA,
        ],
    )
    def k(msg_ref, hid_ref, tidx_ref, seq_ref, h0_ref,
          idx_v, rows_v, hrows_v, h0_v, sem):
        wid = lax.axis_index("s") * 2 + lax.axis_index("c")
        base = wid * _SPW
        pltpu.sync_copy(tidx_ref.at[pl.ds(base, _SPW)], idx_v)
        handles = [
            pltpu.async_copy(
                msg_ref.at[idx_v.at[pl.ds(g * _SGB, _SGB)]],
                rows_v.at[pl.ds(g * _SGB, _SGB)], sem)
            for g in range(_NSG)
        ]

        @pl.when(wid < NM // _MPW)
        def _():
            pltpu.sync_copy(hid_ref.at[pl.ds(wid * (_MPW * L), _MPW * L + 8)],
                            hrows_v)
            for mi in range(_MPW):
                for c in range(H // 16):
                    sl = pl.ds(c * 16, 16)

                    def red(j, m, mi=mi, sl=sl):
                        return jnp.maximum(m, hrows_v[1 + mi * L + j, sl])

                    h0_v[mi, sl] = lax.fori_loop(
                        1, L, red, hrows_v[1 + mi * L, sl])
            pltpu.sync_copy(h0_v, h0_ref.at[pl.ds(wid * _MPW, _MPW)])

        @pl.when((wid >= NM // _MPW) & (wid < NM // _MPW + (NMP - NM) // _MPW))
        def _():
            for mi in range(_MPW):
                for c in range(H // 16):
                    h0_v[mi, pl.ds(c * 16, 16)] = jnp.zeros((16,), jnp.float32)
            pltpu.sync_copy(
                h0_v, h0_ref.at[pl.ds(NM + (wid - NM // _MPW) * _MPW, _MPW)])

        for h in handles:
            h.wait()
        pltpu.sync_copy(rows_v, seq_ref.at[pl.ds(base, _SPW)])

    return k(message, hidden, tidx)


def _gru_kernel(seq, h0p, m0, wif, whf, bif, bhf, wib, whb, bib, bhb, wo1, wo2, bo):
    def body(seq_ref, h0_ref, m0_ref, wif_ref, whf_ref, bif_ref, bhf_ref,
             wib_ref, whb_ref, bib_ref, bhb_ref, wo1_ref, wo2_ref, bo_ref,
             out_ref, r0_ref):
        out_ref[...] = jnp.zeros_like(out_ref)
        wifv = wif_ref[...]
        whfv = whf_ref[...]
        wibv = wib_ref[...]
        whbv = whb_ref[...]
        bifv = bif_ref[...]
        bhfv = bhf_ref[...]
        bibv = bib_ref[...]
        bhbv = bhb_ref[...]
        wo1v = wo1_ref[...]
        wo2v = wo2_ref[...]

        def step(h, x, wih, whh, bi, bh):
            gi = jnp.dot(x, wih, preferred_element_type=jnp.float32) + bi
            gh = jnp.dot(h, whh, preferred_element_type=jnp.float32) + bh
            r = jax.nn.sigmoid(gi[:, :H] + gh[:, :H])
            z = jax.nn.sigmoid(gi[:, H:2 * H] + gh[:, H:2 * H])
            n = jnp.tanh(gi[:, 2 * H:] + r * gh[:, 2 * H:])
            return (1.0 - z) * n + z * h

        def loop(t, carry):
            hf, hb = carry
            xf = seq_ref[pl.ds(t * NMP, NMP), :]
            xb = seq_ref[pl.ds((L - 1 - t) * NMP, NMP), :]
            hf = step(hf, xf, wifv, whfv, bifv, bhfv)
            hb = step(hb, xb, wibv, whbv, bibv, bhbv)
            out_ref[pl.ds(t * NMP, NMP), :] += jnp.dot(
                hf, wo1v, preferred_element_type=jnp.float32)
            out_ref[pl.ds((L - 1 - t) * NMP, NMP), :] += jnp.dot(
                hb, wo2v, preferred_element_type=jnp.float32)
            return (hf, hb)

        h0 = h0_ref[...]
        lax.fori_loop(0, L, loop, (h0, h0))
        out_ref[...] = jnp.maximum(out_ref[...] + bo_ref[...], 0.0)
        r0_ref[...] = jnp.maximum(
            jnp.dot(m0_ref[...], wo1v + wo2v, preferred_element_type=jnp.float32)
            + bo_ref[...], 0.0)

    return pl.pallas_call(
        body,
        out_shape=[jax.ShapeDtypeStruct((L * NMP, H), jnp.float32),
                   jax.ShapeDtypeStruct((8, H), jnp.float32)],
    )(seq, h0p, m0, wif, whf, bif, bhf, wib, whb, bib, bhb, wo1, wo2, bo)


def _sc_mesh():
    return plsc.VectorSubcoreMesh(core_axis_name="c", subcore_axis_name="s")


def _sc_agg(mb, a2b_flat):
    @functools.partial(
        pl.kernel,
        mesh=_sc_mesh(),
        out_type=jax.ShapeDtypeStruct((A_PAD, H), jnp.float32),
        scratch_types=[
            pltpu.VMEM((_APW * MAXB,), jnp.int32),
            pltpu.VMEM((_NBUF, _GA * MAXB, H), jnp.float32),
            pltpu.VMEM((_APW, H), jnp.float32),
            pltpu.SemaphoreType.DMA,
        ],
    )
    def k(mb_ref, idx_ref, out_ref, idx_v, rows_v, out_v, sem):
        wid = lax.axis_index("s") * 2 + lax.axis_index("c")
        pltpu.sync_copy(idx_ref.at[pl.ds(wid * (_APW * MAXB), _APW * MAXB)], idx_v)

        def batch(k_, carry):
            g0 = k_ * _NBUF
            handles = [
                pltpu.async_copy(
                    mb_ref.at[idx_v.at[pl.ds((g0 + b) * (_GA * MAXB), _GA * MAXB)]],
                    rows_v.at[b], sem)
                for b in range(_NBUF)
            ]
            for h in handles:
                h.wait()
            for b in range(_NBUF):
                def atom(i, c2, b=b):
                    base = i * MAXB
                    for c in range(H // 16):
                        sl = pl.ds(c * 16, 16)
                        v = rows_v[b, base, sl]
                        s = v
                        m = v
                        for j in range(1, MAXB):
                            v = rows_v[b, base + j, sl]
                            s = s + v
                            m = jnp.maximum(m, v)
                        out_v[(g0 + b) * _GA + i, sl] = s * m
                    return c2

                lax.fori_loop(0, _GA, atom, 0)
            return carry

        lax.fori_loop(0, _NGA // _NBUF, batch, 0)
        pltpu.sync_copy(out_v, out_ref.at[pl.ds(wid * _APW, _APW)])

    return k(mb, a2b_flat)


def _sc_pre(t_tab, u_tab, b2a, b2revb):
    @functools.partial(
        pl.kernel,
        mesh=_sc_mesh(),
        out_type=jax.ShapeDtypeStruct((E, H), jnp.float32),
        scratch_types=[
            pltpu.VMEM((_EPW,), jnp.int32),
            pltpu.VMEM((_EPW,), jnp.int32),
            pltpu.VMEM((2, _GB, H), jnp.float32),
            pltpu.VMEM((2, _GB, H), jnp.float32),
            pltpu.VMEM((2, _GB, H), jnp.float32),
            pltpu.SemaphoreType.DMA,
        ],
    )
    def k(t_ref, u_ref, ba_ref, br_ref, out_ref,
          ia_v, ir_v, t_v, u_v, o_v, sem):
        wid = lax.axis_index("s") * 2 + lax.axis_index("c")
        base = wid * _EPW
        pltpu.sync_copy(ba_ref.at[pl.ds(base, _EPW)], ia_v)
        pltpu.sync_copy(br_ref.at[pl.ds(base, _EPW)], ir_v)

        def compute_rows(s_, n_rows):
            def row(r, c2):
                for cc in range(H // 16):
                    sl = pl.ds(cc * 16, 16)
                    o_v[s_, r, sl] = t_v[s_, r, sl] - u_v[s_, r, sl]
                return c2

            lax.fori_loop(0, n_rows, row, 0)

        def batch(k_, carry):
            g0 = k_ * 2
            handles = []
            for s_ in range(2):
                g = g0 + s_
                handles.append(pltpu.async_copy(
                    t_ref.at[ia_v.at[pl.ds(g * _GB, _GB)]], t_v.at[s_], sem))
                handles.append(pltpu.async_copy(
                    u_ref.at[ir_v.at[pl.ds(g * _GB, _GB)]], u_v.at[s_], sem))
            for h in handles:
                h.wait()
            for s_ in range(2):
                compute_rows(s_, _GB)
                pltpu.sync_copy(o_v.at[s_],
                                out_ref.at[pl.ds(base + (g0 + s_) * _GB, _GB)])
            return carry

        lax.fori_loop(0, _NGB // 2, batch, 0)

        roff = _NGB * _GB
        ct = pltpu.async_copy(
            t_ref.at[ia_v.at[pl.ds(roff, _GBR)]], t_v.at[0, pl.ds(0, _GBR)], sem)
        cu = pltpu.async_copy(
            u_ref.at[ir_v.at[pl.ds(roff, _GBR)]], u_v.at[0, pl.ds(0, _GBR)], sem)
        ct.wait()
        cu.wait()
        compute_rows(0, _GBR)
        pltpu.sync_copy(o_v.at[0, pl.ds(0, _GBR)],
                        out_ref.at[pl.ds(base + roff, _GBR)])

    return k(t_tab, u_tab, b2a, b2revb)


def kernel(f_atoms, f_bonds, a2b, b2a, b2revb, a_scope, W_i_atom, W_i_bond,
           W_h0, W_h1, W_lr, gru_bias, Wih_f, Whh_f, bih_f, bhh_f,
           Wih_b, Whh_b, bih_b, bhh_b, W_o, b_o):
    del a_scope
    f_atoms_p = jnp.pad(f_atoms, ((0, A_PAD - A), (0, 0)))
    pad_idx = (jnp.arange((A_PAD - A) * MAXB, dtype=jnp.int32) * 997) % E
    a2b_flat = jnp.concatenate([a2b.reshape(-1), pad_idx])

    input_atom = _mm(f_atoms_p, W_i_atom.T, 1024, True)
    input_bond = _mm_t(f_bonds.T, W_i_bond.T, 1280, True)

    msg_atom = input_atom
    mb = input_bond
    for W_h in (W_h0, W_h1):
        agg = _sc_agg(mb, a2b_flat)
        msg_atom = _add(msg_atom, agg)
        pre = _sc_pre(msg_atom, mb, b2a, b2revb)
        mb = _bond_mm(pre, input_bond, W_h.T)

    agg3 = _sc_agg(mb, a2b_flat)
    w_lr_t = W_lr.T
    hidden, message = _mm3(agg3, msg_atom, input_atom,
                           w_lr_t[:H], w_lr_t[H:2 * H], w_lr_t[2 * H:],
                           gru_bias.reshape(1, H))

    tt = jnp.arange(L * NMP, dtype=jnp.int32)
    tidx = 1 + (tt % NMP % NM) * L + tt // NMP
    seq, h0p = _sc_seq(message, hidden, tidx)

    wot = W_o.T
    out_main, row0 = _gru_kernel(
        seq, h0p, message[0:8],
        Wih_f.T, Whh_f.T, bih_f.reshape(1, -1), bhh_f.reshape(1, -1),
        Wih_b.T, Whh_b.T, bih_b.reshape(1, -1), bhh_b.reshape(1, -1),
        wot[:H], wot[H:], b_o.reshape(1, H))

    og = out_main.reshape(L, NMP, H)[:, :NM, :]
    og = jnp.transpose(og, (1, 0, 2)).reshape(NM * L, H)
    return jnp.concatenate([row0[0:1], og], axis=0)


# speedup vs baseline: 1.7172x; 1.0913x over previous
import functools

import jax
import jax.numpy as jnp
from jax import lax
from jax.experimental import pallas as pl
from jax.experimental.pallas import tpu as pltpu
from jax.experimental.pallas import tpu_sc as plsc

A = 10001
E = 320000
MAXB = 32
H = 128
NM = 100
L = 100
A_PAD = 10240
NMP = 128

_NW = 32
_APW = A_PAD // _NW
_GA = 4
_NGA = _APW // _GA
_EPW = E // _NW
_GB = 128
_NGB = _EPW // _GB
_GBR = _EPW - _NGB * _GB
_NBUF = 4


def _mm(x, wt, block_rows, relu):
    n, k = x.shape
    h = wt.shape[1]

    def body(x_ref, w_ref, o_ref):
        y = jnp.dot(x_ref[...], w_ref[...], preferred_element_type=jnp.float32)
        if relu:
            y = jnp.maximum(y, 0.0)
        o_ref[...] = y

    return pl.pallas_call(
        body,
        grid=(n // block_rows,),
        in_specs=[pl.BlockSpec((block_rows, k), lambda i: (i, 0)),
                  pl.BlockSpec((k, h), lambda i: (0, 0))],
        out_specs=pl.BlockSpec((block_rows, h), lambda i: (i, 0)),
        out_shape=jax.ShapeDtypeStruct((n, h), jnp.float32),
    )(x, wt)


def _mm_t(xt, wt, block_cols, relu):
    k, n = xt.shape
    h = wt.shape[1]

    def body(x_ref, w_ref, o_ref):
        y = lax.dot_general(x_ref[...], w_ref[...],
                            (((0,), (0,)), ((), ())),
                            preferred_element_type=jnp.float32)
        if relu:
            y = jnp.maximum(y, 0.0)
        o_ref[...] = y

    return pl.pallas_call(
        body,
        grid=(n // block_cols,),
        in_specs=[pl.BlockSpec((k, block_cols), lambda i: (0, i)),
                  pl.BlockSpec((k, h), lambda i: (0, 0))],
        out_specs=pl.BlockSpec((block_cols, h), lambda i: (i, 0)),
        out_shape=jax.ShapeDtypeStruct((n, h), jnp.float32),
    )(xt, wt)


def _add(msg, agg):
    blk = 1024

    def body(m_ref, a_ref, o_ref):
        o_ref[...] = m_ref[...] + a_ref[...]

    return pl.pallas_call(
        body,
        grid=(A_PAD // blk,),
        in_specs=[pl.BlockSpec((blk, H), lambda i: (i, 0)),
                  pl.BlockSpec((blk, H), lambda i: (i, 0))],
        out_specs=pl.BlockSpec((blk, H), lambda i: (i, 0)),
        out_shape=jax.ShapeDtypeStruct((A_PAD, H), jnp.float32),
    )(msg, agg)


def _bond_mm_chunk(pre_c, ib, wt, e0, carry=None):
    blk = 1280
    esz = pre_c.shape[0]
    off = e0 // blk

    def body(p_ref, i_ref, w_ref, *refs):
        o_ref = refs[-1]
        o_ref[...] = jnp.maximum(
            i_ref[...] + jnp.dot(p_ref[...], w_ref[...],
                                 preferred_element_type=jnp.float32), 0.0)

    in_specs = [pl.BlockSpec((blk, H), lambda i: (i, 0)),
                pl.BlockSpec((blk, H), lambda i: (i + off, 0)),
                pl.BlockSpec((H, H), lambda i: (0, 0))]
    args = [pre_c, ib, wt]
    io_alias = {}
    if carry is not None:
        in_specs.append(pl.BlockSpec(memory_space=pl.ANY))
        args.append(carry)
        io_alias = {3: 0}
    return pl.pallas_call(
        body,
        grid=(esz // blk,),
        in_specs=in_specs,
        out_specs=pl.BlockSpec((blk, H), lambda i: (i + off, 0)),
        out_shape=jax.ShapeDtypeStruct((E, H), jnp.float32),
        input_output_aliases=io_alias,
    )(*args)


def _mm3(agg3, msg2, ia, w0, w1, w2, bias):
    blk = 1024

    def body(a_ref, m_ref, i_ref, w0_ref, w1_ref, w2_ref, b_ref, hid_ref, msg_ref):
        h = (jnp.dot(a_ref[...], w0_ref[...], preferred_element_type=jnp.float32)
             + jnp.dot(m_ref[...], w1_ref[...], preferred_element_type=jnp.float32)
             + jnp.dot(i_ref[...], w2_ref[...], preferred_element_type=jnp.float32))
        hid_ref[...] = h
        msg_ref[...] = jnp.maximum(h + b_ref[...], 0.0)

    return pl.pallas_call(
        body,
        grid=(A_PAD // blk,),
        in_specs=[pl.BlockSpec((blk, H), lambda i: (i, 0)),
                  pl.BlockSpec((blk, H), lambda i: (i, 0)),
                  pl.BlockSpec((blk, H), lambda i: (i, 0)),
                  pl.BlockSpec((H, H), lambda i: (0, 0)),
                  pl.BlockSpec((H, H), lambda i: (0, 0)),
                  pl.BlockSpec((H, H), lambda i: (0, 0)),
                  pl.BlockSpec((1, H), lambda i: (0, 0))],
        out_specs=[pl.BlockSpec((blk, H), lambda i: (i, 0)),
                   pl.BlockSpec((blk, H), lambda i: (i, 0))],
        out_shape=[jax.ShapeDtypeStruct((A_PAD, H), jnp.float32),
                   jax.ShapeDtypeStruct((A_PAD, H), jnp.float32)],
    )(agg3, msg2, ia, w0, w1, w2, bias)


_SPW = (L * NMP) // _NW
_SGB = 80
_NSG = _SPW // _SGB
_MPW = 4


def _sc_seq(message, hidden, tidx):
    @functools.partial(
        pl.kernel,
        mesh=_sc_mesh(),
        out_type=[jax.ShapeDtypeStruct((L * NMP, H), jnp.float32),
                  jax.ShapeDtypeStruct((NMP, H), jnp.float32)],
        scratch_types=[
            pltpu.VMEM((_SPW,), jnp.int32),
            pltpu.VMEM((_SPW, H), jnp.float32),
            pltpu.VMEM((_MPW * L + 8, H), jnp.float32),
            pltpu.VMEM((_MPW, H), jnp.float32),
            pltpu.SemaphoreType.DMA,
        ],
    )
    def k(msg_ref, hid_ref, tidx_ref, seq_ref, h0_ref,
          idx_v, rows_v, hrows_v, h0_v, sem):
        wid = lax.axis_index("s") * 2 + lax.axis_index("c")
        base = wid * _SPW
        pltpu.sync_copy(tidx_ref.at[pl.ds(base, _SPW)], idx_v)
        handles = [
            pltpu.async_copy(
                msg_ref.at[idx_v.at[pl.ds(g * _SGB, _SGB)]],
                rows_v.at[pl.ds(g * _SGB, _SGB)], sem)
            for g in range(_NSG)
        ]

        @pl.when(wid < NM // _MPW)
        def _():
            pltpu.sync_copy(hid_ref.at[pl.ds(wid * (_MPW * L), _MPW * L + 8)],
                            hrows_v)
            for mi in range(_MPW):
                for c in range(H // 16):
                    sl = pl.ds(c * 16, 16)

                    def red(j, m, mi=mi, sl=sl):
                        return jnp.maximum(m, hrows_v[1 + mi * L + j, sl])

                    h0_v[mi, sl] = lax.fori_loop(
                        1, L, red, hrows_v[1 + mi * L, sl])
            pltpu.sync_copy(h0_v, h0_ref.at[pl.ds(wid * _MPW, _MPW)])

        @pl.when((wid >= NM // _MPW) & (wid < NM // _MPW + (NMP - NM) // _MPW))
        def _():
            for mi in range(_MPW):
                for c in range(H // 16):
                    h0_v[mi, pl.ds(c * 16, 16)] = jnp.zeros((16,), jnp.float32)
            pltpu.sync_copy(
                h0_v, h0_ref.at[pl.ds(NM + (wid - NM // _MPW) * _MPW, _MPW)])

        for h in handles:
            h.wait()
        pltpu.sync_copy(rows_v, seq_ref.at[pl.ds(base, _SPW)])

    return k(message, hidden, tidx)


def _gru_kernel(seq, h0p, m0, wif, whf, bif, bhf, wib, whb, bib, bhb, wo1, wo2, bo):
    def body(seq_ref, h0_ref, m0_ref, wif_ref, whf_ref, bif_ref, bhf_ref,
             wib_ref, whb_ref, bib_ref, bhb_ref, wo1_ref, wo2_ref, bo_ref,
             out_ref, r0_ref):
        out_ref[...] = jnp.zeros_like(out_ref)
        wifv = wif_ref[...]
        whfv = whf_ref[...]
        wibv = wib_ref[...]
        whbv = whb_ref[...]
        bifv = bif_ref[...]
        bhfv = bhf_ref[...]
        bibv = bib_ref[...]
        bhbv = bhb_ref[...]
        wo1v = wo1_ref[...]
        wo2v = wo2_ref[...]

        def step(h, x, wih, whh, bi, bh):
            gi = jnp.dot(x, wih, preferred_element_type=jnp.float32) + bi
            gh = jnp.dot(h, whh, preferred_element_type=jnp.float32) + bh
            r = jax.nn.sigmoid(gi[:, :H] + gh[:, :H])
            z = jax.nn.sigmoid(gi[:, H:2 * H] + gh[:, H:2 * H])
            n = jnp.tanh(gi[:, 2 * H:] + r * gh[:, 2 * H:])
            return (1.0 - z) * n + z * h

        def loop(t, carry):
            hf, hb = carry
            xf = seq_ref[pl.ds(t * NMP, NMP), :]
            xb = seq_ref[pl.ds((L - 1 - t) * NMP, NMP), :]
            hf = step(hf, xf, wifv, whfv, bifv, bhfv)
            hb = step(hb, xb, wibv, whbv, bibv, bhbv)
            out_ref[pl.ds(t * NMP, NMP), :] += jnp.dot(
                hf, wo1v, preferred_element_type=jnp.float32)
            out_ref[pl.ds((L - 1 - t) * NMP, NMP), :] += jnp.dot(
                hb, wo2v, preferred_element_type=jnp.float32)
            return (hf, hb)

        h0 = h0_ref[...]
        lax.fori_loop(0, L, loop, (h0, h0))
        out_ref[...] = jnp.maximum(out_ref[...] + bo_ref[...], 0.0)
        r0_ref[...] = jnp.maximum(
            jnp.dot(m0_ref[...], wo1v + wo2v, preferred_element_type=jnp.float32)
            + bo_ref[...], 0.0)

    return pl.pallas_call(
        body,
        out_shape=[jax.ShapeDtypeStruct((L * NMP, H), jnp.float32),
                   jax.ShapeDtypeStruct((8, H), jnp.float32)],
    )(seq, h0p, m0, wif, whf, bif, bhf, wib, whb, bib, bhb, wo1, wo2, bo)


def _sc_mesh():
    return plsc.VectorSubcoreMesh(core_axis_name="c", subcore_axis_name="s")


def _sc_agg(mb, a2b_flat):
    @functools.partial(
        pl.kernel,
        mesh=_sc_mesh(),
        out_type=jax.ShapeDtypeStruct((A_PAD, H), jnp.float32),
        scratch_types=[
            pltpu.VMEM((_APW * MAXB,), jnp.int32),
            pltpu.VMEM((_NBUF, _GA * MAXB, H), jnp.float32),
            pltpu.VMEM((_APW, H), jnp.float32),
            pltpu.SemaphoreType.DMA,
        ],
    )
    def k(mb_ref, idx_ref, out_ref, idx_v, rows_v, out_v, sem):
        wid = lax.axis_index("s") * 2 + lax.axis_index("c")
        pltpu.sync_copy(idx_ref.at[pl.ds(wid * (_APW * MAXB), _APW * MAXB)], idx_v)

        def batch(k_, carry):
            g0 = k_ * _NBUF
            handles = [
                pltpu.async_copy(
                    mb_ref.at[idx_v.at[pl.ds((g0 + b) * (_GA * MAXB), _GA * MAXB)]],
                    rows_v.at[b], sem)
                for b in range(_NBUF)
            ]
            for h in handles:
                h.wait()
            for b in range(_NBUF):
                def atom(i, c2, b=b):
                    base = i * MAXB
                    for c in range(H // 16):
                        sl = pl.ds(c * 16, 16)
                        v = rows_v[b, base, sl]
                        s = v
                        m = v
                        for j in range(1, MAXB):
                            v = rows_v[b, base + j, sl]
                            s = s + v
                            m = jnp.maximum(m, v)
                        out_v[(g0 + b) * _GA + i, sl] = s * m
                    return c2

                lax.fori_loop(0, _GA, atom, 0)
            return carry

        lax.fori_loop(0, _NGA // _NBUF, batch, 0)
        pltpu.sync_copy(out_v, out_ref.at[pl.ds(wid * _APW, _APW)])

    return k(mb, a2b_flat)


def _sc_pre(t_tab, u_tab, b2a, b2revb, e0, esz):
    epw = esz // _NW
    ngb = epw // _GB
    npair = ngb // 2
    tail = [(g * _GB, _GB) for g in range(2 * npair, ngb)]
    if epw % _GB:
        tail.append((ngb * _GB, epw % _GB))

    @functools.partial(
        pl.kernel,
        mesh=_sc_mesh(),
        out_type=jax.ShapeDtypeStruct((esz, H), jnp.float32),
        scratch_types=[
            pltpu.VMEM((_EPW,), jnp.int32),
            pltpu.VMEM((_EPW,), jnp.int32),
            pltpu.VMEM((2, _GB, H), jnp.float32),
            pltpu.VMEM((2, _GB, H), jnp.float32),
            pltpu.VMEM((2, _GB, H), jnp.float32),
            pltpu.SemaphoreType.DMA,
        ],
    )
    def k(t_ref, u_ref, ba_ref, br_ref, out_ref,
          ia_v, ir_v, t_v, u_v, o_v, sem):
        wid = lax.axis_index("s") * 2 + lax.axis_index("c")
        base = wid * epw
        pltpu.sync_copy(ba_ref.at[pl.ds(e0 + base, epw)],
                        ia_v.at[pl.ds(0, epw)])
        pltpu.sync_copy(br_ref.at[pl.ds(e0 + base, epw)],
                        ir_v.at[pl.ds(0, epw)])

        def compute_rows(s_, n_rows):
            def row(r, c2):
                for cc in range(H // 16):
                    sl = pl.ds(cc * 16, 16)
                    o_v[s_, r, sl] = t_v[s_, r, sl] - u_v[s_, r, sl]
                return c2

            lax.fori_loop(0, n_rows, row, 0)

        def batch(k_, carry):
            g0 = k_ * 2
            handles = []
            for s_ in range(2):
                g = g0 + s_
                handles.append(pltpu.async_copy(
                    t_ref.at[ia_v.at[pl.ds(g * _GB, _GB)]], t_v.at[s_], sem))
                handles.append(pltpu.async_copy(
                    u_ref.at[ir_v.at[pl.ds(g * _GB, _GB)]], u_v.at[s_], sem))
            for h in handles:
                h.wait()
            for s_ in range(2):
                compute_rows(s_, _GB)
                pltpu.sync_copy(o_v.at[s_],
                                out_ref.at[pl.ds(base + (g0 + s_) * _GB, _GB)])
            return carry

        lax.fori_loop(0, npair, batch, 0)

        for roff, rn in tail:
            ct = pltpu.async_copy(
                t_ref.at[ia_v.at[pl.ds(roff, rn)]], t_v.at[0, pl.ds(0, rn)], sem)
            cu = pltpu.async_copy(
                u_ref.at[ir_v.at[pl.ds(roff, rn)]], u_v.at[0, pl.ds(0, rn)], sem)
            ct.wait()
            cu.wait()
            compute_rows(0, rn)
            pltpu.sync_copy(o_v.at[0, pl.ds(0, rn)],
                            out_ref.at[pl.ds(base + roff, rn)])

    return k(t_tab, u_tab, b2a, b2revb)


def kernel(f_atoms, f_bonds, a2b, b2a, b2revb, a_scope, W_i_atom, W_i_bond,
           W_h0, W_h1, W_lr, gru_bias, Wih_f, Whh_f, bih_f, bhh_f,
           Wih_b, Whh_b, bih_b, bhh_b, W_o, b_o):
    del a_scope
    f_atoms_p = jnp.pad(f_atoms, ((0, A_PAD - A), (0, 0)))
    pad_idx = (jnp.arange((A_PAD - A) * MAXB, dtype=jnp.int32) * 997) % E
    a2b_flat = jnp.concatenate([a2b.reshape(-1), pad_idx])

    input_atom = _mm(f_atoms_p, W_i_atom.T, 1024, True)
    input_bond = _mm_t(f_bonds.T, W_i_bond.T, 1280, True)

    msg_atom = input_atom
    mb = input_bond
    e2 = E // 2
    for W_h in (W_h0, W_h1):
        agg = _sc_agg(mb, a2b_flat)
        msg_atom = _add(msg_atom, agg)
        pre_a = _sc_pre(msg_atom, mb, b2a, b2revb, 0, e2)
        pre_b = _sc_pre(msg_atom, mb, b2a, b2revb, e2, e2)
        tmp = _bond_mm_chunk(pre_a, input_bond, W_h.T, 0)
        mb = _bond_mm_chunk(pre_b, input_bond, W_h.T, e2, tmp)

    agg3 = _sc_agg(mb, a2b_flat)
    w_lr_t = W_lr.T
    hidden, message = _mm3(agg3, msg_atom, input_atom,
                           w_lr_t[:H], w_lr_t[H:2 * H], w_lr_t[2 * H:],
                           gru_bias.reshape(1, H))

    tt = jnp.arange(L * NMP, dtype=jnp.int32)
    tidx = 1 + (tt % NMP % NM) * L + tt // NMP
    seq, h0p = _sc_seq(message, hidden, tidx)

    wot = W_o.T
    out_main, row0 = _gru_kernel(
        seq, h0p, message[0:8],
        Wih_f.T, Whh_f.T, bih_f.reshape(1, -1), bhh_f.reshape(1, -1),
        Wih_b.T, Whh_b.T, bih_b.reshape(1, -1), bhh_b.reshape(1, -1),
        wot[:H], wot[H:], b_o.reshape(1, H))

    og = out_main.reshape(L, NMP, H)[:, :NM, :]
    og = jnp.transpose(og, (1, 0, 2)).reshape(NM * L, H)
    return jnp.concatenate([row0[0:1], og], axis=0)


# speedup vs baseline: 1.8574x; 1.0817x over previous
import functools

import jax
import jax.numpy as jnp
from jax import lax
from jax.experimental import pallas as pl
from jax.experimental.pallas import tpu as pltpu
from jax.experimental.pallas import tpu_sc as plsc

A = 10001
E = 320000
MAXB = 32
H = 128
NM = 100
L = 100
A_PAD = 10240
NMP = 128

_NW = 32
_APW = A_PAD // _NW
_GA = 4
_NGA = _APW // _GA
_EPW = E // _NW
_GB = 128
_NGB = _EPW // _GB
_GBR = _EPW - _NGB * _GB
_NBUF = 4


def _mm(x, wt, block_rows, relu):
    n, k = x.shape
    h = wt.shape[1]

    def body(x_ref, w_ref, o_ref):
        y = jnp.dot(x_ref[...], w_ref[...], preferred_element_type=jnp.float32)
        if relu:
            y = jnp.maximum(y, 0.0)
        o_ref[...] = y

    return pl.pallas_call(
        body,
        grid=(n // block_rows,),
        in_specs=[pl.BlockSpec((block_rows, k), lambda i: (i, 0)),
                  pl.BlockSpec((k, h), lambda i: (0, 0))],
        out_specs=pl.BlockSpec((block_rows, h), lambda i: (i, 0)),
        out_shape=jax.ShapeDtypeStruct((n, h), jnp.float32),
    )(x, wt)


def _mm_t(xt, wt, block_cols, relu):
    k, n = xt.shape
    h = wt.shape[1]

    def body(x_ref, w_ref, o_ref):
        y = lax.dot_general(x_ref[...], w_ref[...],
                            (((0,), (0,)), ((), ())),
                            preferred_element_type=jnp.float32)
        if relu:
            y = jnp.maximum(y, 0.0)
        o_ref[...] = y

    return pl.pallas_call(
        body,
        grid=(n // block_cols,),
        in_specs=[pl.BlockSpec((k, block_cols), lambda i: (0, i)),
                  pl.BlockSpec((k, h), lambda i: (0, 0))],
        out_specs=pl.BlockSpec((block_cols, h), lambda i: (i, 0)),
        out_shape=jax.ShapeDtypeStruct((n, h), jnp.float32),
    )(xt, wt)


def _add(msg, agg):
    blk = 1024

    def body(m_ref, a_ref, o_ref):
        o_ref[...] = m_ref[...] + a_ref[...]

    return pl.pallas_call(
        body,
        grid=(A_PAD // blk,),
        in_specs=[pl.BlockSpec((blk, H), lambda i: (i, 0)),
                  pl.BlockSpec((blk, H), lambda i: (i, 0))],
        out_specs=pl.BlockSpec((blk, H), lambda i: (i, 0)),
        out_shape=jax.ShapeDtypeStruct((A_PAD, H), jnp.float32),
    )(msg, agg)


def _bond_mm_chunk(pre_c, ib, wt, e0, carry=None):
    blk = 1280
    esz = pre_c.shape[0]
    off = e0 // blk

    def body(p_ref, i_ref, w_ref, *refs):
        o_ref = refs[-1]
        o_ref[...] = jnp.maximum(
            i_ref[...] + jnp.dot(p_ref[...], w_ref[...],
                                 preferred_element_type=jnp.float32), 0.0)

    in_specs = [pl.BlockSpec((blk, H), lambda i: (i, 0)),
                pl.BlockSpec((blk, H), lambda i: (i + off, 0)),
                pl.BlockSpec((H, H), lambda i: (0, 0))]
    args = [pre_c, ib, wt]
    io_alias = {}
    if carry is not None:
        in_specs.append(pl.BlockSpec(memory_space=pl.ANY))
        args.append(carry)
        io_alias = {3: 0}
    return pl.pallas_call(
        body,
        grid=(esz // blk,),
        in_specs=in_specs,
        out_specs=pl.BlockSpec((blk, H), lambda i: (i + off, 0)),
        out_shape=jax.ShapeDtypeStruct((E, H), jnp.float32),
        input_output_aliases=io_alias,
    )(*args)


def _mm3(agg3, msg2, ia, w0, w1, w2, bias):
    blk = 1024

    def body(a_ref, m_ref, i_ref, w0_ref, w1_ref, w2_ref, b_ref, hid_ref, msg_ref):
        h = (jnp.dot(a_ref[...], w0_ref[...], preferred_element_type=jnp.float32)
             + jnp.dot(m_ref[...], w1_ref[...], preferred_element_type=jnp.float32)
             + jnp.dot(i_ref[...], w2_ref[...], preferred_element_type=jnp.float32))
        hid_ref[...] = h
        msg_ref[...] = jnp.maximum(h + b_ref[...], 0.0)

    return pl.pallas_call(
        body,
        grid=(A_PAD // blk,),
        in_specs=[pl.BlockSpec((blk, H), lambda i: (i, 0)),
                  pl.BlockSpec((blk, H), lambda i: (i, 0)),
                  pl.BlockSpec((blk, H), lambda i: (i, 0)),
                  pl.BlockSpec((H, H), lambda i: (0, 0)),
                  pl.BlockSpec((H, H), lambda i: (0, 0)),
                  pl.BlockSpec((H, H), lambda i: (0, 0)),
                  pl.BlockSpec((1, H), lambda i: (0, 0))],
        out_specs=[pl.BlockSpec((blk, H), lambda i: (i, 0)),
                   pl.BlockSpec((blk, H), lambda i: (i, 0))],
        out_shape=[jax.ShapeDtypeStruct((A_PAD, H), jnp.float32),
                   jax.ShapeDtypeStruct((A_PAD, H), jnp.float32)],
    )(agg3, msg2, ia, w0, w1, w2, bias)


_SPW = (L * NMP) // _NW
_SGB = 80
_NSG = _SPW // _SGB
_MPW = 4


def _sc_seq(message, hidden, tidx):
    @functools.partial(
        pl.kernel,
        mesh=_sc_mesh(),
        out_type=[jax.ShapeDtypeStruct((L * NMP, H), jnp.float32),
                  jax.ShapeDtypeStruct((NMP, H), jnp.float32)],
        scratch_types=[
            pltpu.VMEM((_SPW,), jnp.int32),
            pltpu.VMEM((_SPW, H), jnp.float32),
            pltpu.VMEM((_MPW * L + 8, H), jnp.float32),
            pltpu.VMEM((_MPW, H), jnp.float32),
            pltpu.SemaphoreType.DMA,
        ],
    )
    def k(msg_ref, hid_ref, tidx_ref, seq_ref, h0_ref,
          idx_v, rows_v, hrows_v, h0_v, sem):
        wid = lax.axis_index("s") * 2 + lax.axis_index("c")
        base = wid * _SPW
        pltpu.sync_copy(tidx_ref.at[pl.ds(base, _SPW)], idx_v)
        handles = [
            pltpu.async_copy(
                msg_ref.at[idx_v.at[pl.ds(g * _SGB, _SGB)]],
                rows_v.at[pl.ds(g * _SGB, _SGB)], sem)
            for g in range(_NSG)
        ]

        @pl.when(wid < NM // _MPW)
        def _():
            pltpu.sync_copy(hid_ref.at[pl.ds(wid * (_MPW * L), _MPW * L + 8)],
                            hrows_v)
            for mi in range(_MPW):
                for c in range(H // 16):
                    sl = pl.ds(c * 16, 16)

                    def red(j, m, mi=mi, sl=sl):
                        return jnp.maximum(m, hrows_v[1 + mi * L + j, sl])

                    h0_v[mi, sl] = lax.fori_loop(
                        1, L, red, hrows_v[1 + mi * L, sl])
            pltpu.sync_copy(h0_v, h0_ref.at[pl.ds(wid * _MPW, _MPW)])

        @pl.when((wid >= NM // _MPW) & (wid < NM // _MPW + (NMP - NM) // _MPW))
        def _():
            for mi in range(_MPW):
                for c in range(H // 16):
                    h0_v[mi, pl.ds(c * 16, 16)] = jnp.zeros((16,), jnp.float32)
            pltpu.sync_copy(
                h0_v, h0_ref.at[pl.ds(NM + (wid - NM // _MPW) * _MPW, _MPW)])

        for h in handles:
            h.wait()
        pltpu.sync_copy(rows_v, seq_ref.at[pl.ds(base, _SPW)])

    return k(message, hidden, tidx)


def _gru_kernel(seq, h0p, m0, wif, whf, bif, bhf, wib, whb, bib, bhb, wo1, wo2, bo):
    def body(seq_ref, h0_ref, m0_ref, wif_ref, whf_ref, bif_ref, bhf_ref,
             wib_ref, whb_ref, bib_ref, bhb_ref, wo1_ref, wo2_ref, bo_ref,
             out_ref, r0_ref):
        out_ref[...] = jnp.zeros_like(out_ref)
        wifv = wif_ref[...]
        whfv = whf_ref[...]
        wibv = wib_ref[...]
        whbv = whb_ref[...]
        bifv = bif_ref[...]
        bhfv = bhf_ref[...]
        bibv = bib_ref[...]
        bhbv = bhb_ref[...]
        wo1v = wo1_ref[...]
        wo2v = wo2_ref[...]

        def step(h, x, wih, whh, bi, bh):
            gi = jnp.dot(x, wih, preferred_element_type=jnp.float32) + bi
            gh = jnp.dot(h, whh, preferred_element_type=jnp.float32) + bh
            r = jax.nn.sigmoid(gi[:, :H] + gh[:, :H])
            z = jax.nn.sigmoid(gi[:, H:2 * H] + gh[:, H:2 * H])
            n = jnp.tanh(gi[:, 2 * H:] + r * gh[:, 2 * H:])
            return (1.0 - z) * n + z * h

        def loop(t, carry):
            hf, hb = carry
            xf = seq_ref[pl.ds(t * NMP, NMP), :]
            xb = seq_ref[pl.ds((L - 1 - t) * NMP, NMP), :]
            hf = step(hf, xf, wifv, whfv, bifv, bhfv)
            hb = step(hb, xb, wibv, whbv, bibv, bhbv)
            out_ref[pl.ds(t * NMP, NMP), :] += jnp.dot(
                hf, wo1v, preferred_element_type=jnp.float32)
            out_ref[pl.ds((L - 1 - t) * NMP, NMP), :] += jnp.dot(
                hb, wo2v, preferred_element_type=jnp.float32)
            return (hf, hb)

        h0 = h0_ref[...]
        lax.fori_loop(0, L, loop, (h0, h0))
        out_ref[...] = jnp.maximum(out_ref[...] + bo_ref[...], 0.0)
        r0_ref[...] = jnp.maximum(
            jnp.dot(m0_ref[...], wo1v + wo2v, preferred_element_type=jnp.float32)
            + bo_ref[...], 0.0)

    return pl.pallas_call(
        body,
        out_shape=[jax.ShapeDtypeStruct((L * NMP, H), jnp.float32),
                   jax.ShapeDtypeStruct((8, H), jnp.float32)],
    )(seq, h0p, m0, wif, whf, bif, bhf, wib, whb, bib, bhb, wo1, wo2, bo)


def _sc_mesh():
    return plsc.VectorSubcoreMesh(core_axis_name="c", subcore_axis_name="s")


def _sc_agg(mb, a2b_flat):
    @functools.partial(
        pl.kernel,
        mesh=_sc_mesh(),
        out_type=jax.ShapeDtypeStruct((A_PAD, H), jnp.float32),
        scratch_types=[
            pltpu.VMEM((_APW * MAXB,), jnp.int32),
            pltpu.VMEM((_NBUF, _GA * MAXB, H), jnp.float32),
            pltpu.VMEM((_APW, H), jnp.float32),
            pltpu.SemaphoreType.DMA,
        ],
    )
    def k(mb_ref, idx_ref, out_ref, idx_v, rows_v, out_v, sem):
        wid = lax.axis_index("s") * 2 + lax.axis_index("c")
        pltpu.sync_copy(idx_ref.at[pl.ds(wid * (_APW * MAXB), _APW * MAXB)], idx_v)

        def batch(k_, carry):
            g0 = k_ * _NBUF
            handles = [
                pltpu.async_copy(
                    mb_ref.at[idx_v.at[pl.ds((g0 + b) * (_GA * MAXB), _GA * MAXB)]],
                    rows_v.at[b], sem)
                for b in range(_NBUF)
            ]
            for h in handles:
                h.wait()
            for b in range(_NBUF):
                def atom(i, c2, b=b):
                    base = i * MAXB
                    for c in range(H // 16):
                        sl = pl.ds(c * 16, 16)
                        v = rows_v[b, base, sl]
                        s = v
                        m = v
                        for j in range(1, MAXB):
                            v = rows_v[b, base + j, sl]
                            s = s + v
                            m = jnp.maximum(m, v)
                        out_v[(g0 + b) * _GA + i, sl] = s * m
                    return c2

                lax.fori_loop(0, _GA, atom, 0)
            return carry

        lax.fori_loop(0, _NGA // _NBUF, batch, 0)
        pltpu.sync_copy(out_v, out_ref.at[pl.ds(wid * _APW, _APW)])

    return k(mb, a2b_flat)


def _sc_pre(t_tab, u_tab, b2a, b2revb, e0, esz):
    epw = esz // _NW
    ngb = epw // _GB
    npair = ngb // 2
    tail = [(g * _GB, _GB) for g in range(2 * npair, ngb)]
    if epw % _GB:
        tail.append((ngb * _GB, epw % _GB))

    @functools.partial(
        pl.kernel,
        mesh=_sc_mesh(),
        out_type=jax.ShapeDtypeStruct((esz, H), jnp.float32),
        scratch_types=[
            pltpu.VMEM((_EPW,), jnp.int32),
            pltpu.VMEM((_EPW,), jnp.int32),
            pltpu.VMEM((2, _GB, H), jnp.float32),
            pltpu.VMEM((2, _GB, H), jnp.float32),
            pltpu.VMEM((2, _GB, H), jnp.float32),
            pltpu.SemaphoreType.DMA,
        ],
    )
    def k(t_ref, u_ref, ba_ref, br_ref, out_ref,
          ia_v, ir_v, t_v, u_v, o_v, sem):
        wid = lax.axis_index("s") * 2 + lax.axis_index("c")
        base = wid * epw
        pltpu.sync_copy(ba_ref.at[pl.ds(e0 + base, epw)],
                        ia_v.at[pl.ds(0, epw)])
        pltpu.sync_copy(br_ref.at[pl.ds(e0 + base, epw)],
                        ir_v.at[pl.ds(0, epw)])

        def compute_rows(s_, n_rows):
            def row(r, c2):
                for cc in range(H // 16):
                    sl = pl.ds(cc * 16, 16)
                    o_v[s_, r, sl] = t_v[s_, r, sl] - u_v[s_, r, sl]
                return c2

            lax.fori_loop(0, n_rows, row, 0)

        def batch(k_, carry):
            g0 = k_ * 2
            handles = []
            for s_ in range(2):
                g = g0 + s_
                handles.append(pltpu.async_copy(
                    t_ref.at[ia_v.at[pl.ds(g * _GB, _GB)]], t_v.at[s_], sem))
                handles.append(pltpu.async_copy(
                    u_ref.at[ir_v.at[pl.ds(g * _GB, _GB)]], u_v.at[s_], sem))
            for h in handles:
                h.wait()
            for s_ in range(2):
                compute_rows(s_, _GB)
                pltpu.sync_copy(o_v.at[s_],
                                out_ref.at[pl.ds(base + (g0 + s_) * _GB, _GB)])
            return carry

        lax.fori_loop(0, npair, batch, 0)

        for roff, rn in tail:
            ct = pltpu.async_copy(
                t_ref.at[ia_v.at[pl.ds(roff, rn)]], t_v.at[0, pl.ds(0, rn)], sem)
            cu = pltpu.async_copy(
                u_ref.at[ir_v.at[pl.ds(roff, rn)]], u_v.at[0, pl.ds(0, rn)], sem)
            ct.wait()
            cu.wait()
            compute_rows(0, rn)
            pltpu.sync_copy(o_v.at[0, pl.ds(0, rn)],
                            out_ref.at[pl.ds(base + roff, rn)])

    return k(t_tab, u_tab, b2a, b2revb)


def kernel(f_atoms, f_bonds, a2b, b2a, b2revb, a_scope, W_i_atom, W_i_bond,
           W_h0, W_h1, W_lr, gru_bias, Wih_f, Whh_f, bih_f, bhh_f,
           Wih_b, Whh_b, bih_b, bhh_b, W_o, b_o):
    del a_scope
    f_atoms_p = jnp.pad(f_atoms, ((0, A_PAD - A), (0, 0)))
    pad_idx = (jnp.arange((A_PAD - A) * MAXB, dtype=jnp.int32) * 997) % E
    a2b_flat = jnp.concatenate([a2b.reshape(-1), pad_idx])

    input_atom = _mm(f_atoms_p, W_i_atom.T, 1024, True)
    input_bond = _mm_t(f_bonds.T, W_i_bond.T, 2560, True)

    msg_atom = input_atom
    mb = input_bond
    sizes = [63 * 1280, 63 * 1280, 62 * 1280, 62 * 1280]
    offs = [0, sizes[0], sizes[0] + sizes[1], sizes[0] + sizes[1] + sizes[2]]
    for W_h in (W_h0, W_h1):
        agg = _sc_agg(mb, a2b_flat)
        msg_atom = _add(msg_atom, agg)
        tmp = None
        pre_prev = _sc_pre(msg_atom, mb, b2a, b2revb, offs[0], sizes[0])
        for ci in range(4):
            pre_next = (_sc_pre(msg_atom, mb, b2a, b2revb,
                                offs[ci + 1], sizes[ci + 1])
                        if ci + 1 < 4 else None)
            tmp = _bond_mm_chunk(pre_prev, input_bond, W_h.T, offs[ci], tmp)
            pre_prev = pre_next
        mb = tmp

    agg3 = _sc_agg(mb, a2b_flat)
    w_lr_t = W_lr.T
    hidden, message = _mm3(agg3, msg_atom, input_atom,
                           w_lr_t[:H], w_lr_t[H:2 * H], w_lr_t[2 * H:],
                           gru_bias.reshape(1, H))

    tt = jnp.arange(L * NMP, dtype=jnp.int32)
    tidx = 1 + (tt % NMP % NM) * L + tt // NMP
    seq, h0p = _sc_seq(message, hidden, tidx)

    wot = W_o.T
    out_main, row0 = _gru_kernel(
        seq, h0p, message[0:8],
        Wih_f.T, Whh_f.T, bih_f.reshape(1, -1), bhh_f.reshape(1, -1),
        Wih_b.T, Whh_b.T, bih_b.reshape(1, -1), bhh_b.reshape(1, -1),
        wot[:H], wot[H:], b_o.reshape(1, H))

    og = out_main.reshape(L, NMP, H)[:, :NM, :]
    og = jnp.transpose(og, (1, 0, 2)).reshape(NM * L, H)
    return jnp.concatenate([row0[0:1], og], axis=0)
